# baseline (speedup 1.0000x reference)
import functools

import jax
import jax.numpy as jnp
from jax import lax
from jax.experimental import pallas as pl
from jax.experimental.pallas import tpu as pltpu
from jax.experimental.pallas import tpu_sc as plsc

N = 10000
E = 320000
D = 128

NC = 2
NS = 16
NW = NC * NS
CHUNK = 128
CPT = 80
STEAL = 16
EPT = CHUNK * CPT
E_PAD = EPT * NW
N_PAD = 10112
RPT = N_PAD // NS


def _sc_agg(x, src_t, dst_t, zeros):
    mesh = plsc.VectorSubcoreMesh(core_axis_name="c", subcore_axis_name="s")

    @functools.partial(
        pl.kernel,
        out_type=jax.ShapeDtypeStruct((NC, N_PAD, D), jnp.float32),
        mesh=mesh,
        scratch_types=[
            pltpu.VMEM((CPT, CHUNK), jnp.int32),
            pltpu.VMEM((CPT, CHUNK), jnp.int32),
            pltpu.VMEM((STEAL, CHUNK), jnp.int32),
            pltpu.VMEM((STEAL, CHUNK), jnp.int32),
            pltpu.VMEM((CHUNK, D), jnp.float32),
            pltpu.VMEM_SHARED((N_PAD, D), jnp.float32),
            pltpu.SemaphoreType.DMA,
        ],
    )
    def k(x_hbm, src_hbm, dst_hbm, z_hbm, out_hbm,
          src_v, dst_v, srcx_v, dstx_v, rows_v, agg_s, sem):
        cid = lax.axis_index("c")
        sid = lax.axis_index("s")
        wid = sid * NC + cid
        r0 = sid * RPT
        pltpu.sync_copy(z_hbm.at[pl.ds(r0, RPT)], agg_s.at[pl.ds(r0, RPT)])
        pltpu.sync_copy(src_hbm.at[wid], src_v)
        pltpu.sync_copy(dst_hbm.at[wid], dst_v)

        @pl.when(cid == 1)
        def _():
            pltpu.sync_copy(src_hbm.at[wid - 1, pl.ds(CPT - STEAL, STEAL)],
                            srcx_v)
            pltpu.sync_copy(dst_hbm.at[wid - 1, pl.ds(CPT - STEAL, STEAL)],
                            dstx_v)

        plsc.subcore_barrier()

        @pl.loop(0, CPT - STEAL)
        def _(j):
            pltpu.async_copy(x_hbm.at[src_v.at[j]], rows_v, sem).wait()
            pltpu.sync_copy(rows_v, agg_s.at[dst_v.at[j]], add=True)

        @pl.when(cid == 1)
        def _():
            @pl.loop(CPT - STEAL, CPT)
            def _(j):
                pltpu.async_copy(x_hbm.at[src_v.at[j]], rows_v, sem).wait()
                pltpu.sync_copy(rows_v, agg_s.at[dst_v.at[j]], add=True)

            @pl.loop(0, STEAL)
            def _(j):
                pltpu.async_copy(x_hbm.at[srcx_v.at[j]], rows_v, sem).wait()
                pltpu.sync_copy(rows_v, agg_s.at[dstx_v.at[j]], add=True)

        plsc.subcore_barrier()
        pltpu.sync_copy(agg_s.at[pl.ds(r0, RPT)],
                        out_hbm.at[cid, pl.ds(r0, RPT)])

    return k(x, src_t, dst_t, zeros)


def _tc_layer(p0, p1, x, W_relT, b8, W_rootT, do_relu):
    BLK = 1000

    def body(p0_ref, p1_ref, x_ref, wr_ref, b_ref, wt_ref, o_ref):
        agg = p0_ref[...] + p1_ref[...]
        acc = jnp.dot(agg, wr_ref[...], preferred_element_type=jnp.float32)
        acc = acc + jnp.dot(x_ref[...], wt_ref[...],
                            preferred_element_type=jnp.float32)
        acc = acc + b_ref[0:1, :]
        if do_relu:
            acc = jnp.maximum(acc, 0.0)
        o_ref[...] = acc

    return pl.pallas_call(
        body,
        grid=(N // BLK,),
        in_specs=[
            pl.BlockSpec((BLK, D), lambda i: (i, 0)),
            pl.BlockSpec((BLK, D), lambda i: (i, 0)),
            pl.BlockSpec((BLK, D), lambda i: (i, 0)),
            pl.BlockSpec((D, D), lambda i: (0, 0)),
            pl.BlockSpec((8, D), lambda i: (0, 0)),
            pl.BlockSpec((D, D), lambda i: (0, 0)),
        ],
        out_specs=pl.BlockSpec((BLK, D), lambda i: (i, 0)),
        out_shape=jax.ShapeDtypeStruct((N, D), jnp.float32),
    )(p0, p1, x, W_relT, b8, W_rootT)


def kernel(h, edge_index, edge_attr, W_rel1, b_rel1, W_root1, W_rel2, b_rel2, W_root2):
    src = edge_index[0]
    dst = edge_index[1]
    pad = E_PAD - E
    src_t = jnp.concatenate(
        [src, jnp.zeros((pad,), jnp.int32)]).reshape(NW, CPT, CHUNK)
    dst_t = jnp.concatenate(
        [dst, jnp.full((pad,), N, jnp.int32)]).reshape(NW, CPT, CHUNK)
    zeros = jnp.zeros((N_PAD, D), jnp.float32)
    b1 = jnp.broadcast_to(b_rel1.reshape(1, D), (8, D))
    b2 = jnp.broadcast_to(b_rel2.reshape(1, D), (8, D))

    p = _sc_agg(h, src_t, dst_t, zeros)
    x1 = _tc_layer(p[0, :N], p[1, :N], h, W_rel1.T, b1, W_root1.T, True)
    p2 = _sc_agg(x1, src_t, dst_t, zeros)
    out = _tc_layer(p2[0, :N], p2[1, :N], x1, W_rel2.T, b2, W_root2.T, False)
    return out


# speedup vs baseline: 1.2549x; 1.2549x over previous
import functools

import jax
import jax.numpy as jnp
from jax import lax
from jax.experimental import pallas as pl
from jax.experimental.pallas import tpu as pltpu
from jax.experimental.pallas import tpu_sc as plsc

N = 10000
E = 320000
D = 128

NC = 2
NS = 16
NW = NC * NS
CHUNK = 128
CPT = 80
STEAL = 16
EPT = CHUNK * CPT
E_PAD = EPT * NW
N_PAD = 10112
RPT = N_PAD // NS


def _sc_agg(x, src_t, dst_t, zeros):
    mesh = plsc.VectorSubcoreMesh(core_axis_name="c", subcore_axis_name="s")

    @functools.partial(
        pl.kernel,
        out_type=jax.ShapeDtypeStruct((NC, N_PAD, D), jnp.float32),
        mesh=mesh,
        scratch_types=[
            pltpu.VMEM((CPT, CHUNK), jnp.int32),
            pltpu.VMEM((CPT, CHUNK), jnp.int32),
            pltpu.VMEM((STEAL, CHUNK), jnp.int32),
            pltpu.VMEM((STEAL, CHUNK), jnp.int32),
            pltpu.VMEM((CHUNK, D), jnp.float32),
            pltpu.VMEM_SHARED((N_PAD, D), jnp.float32),
            pltpu.SemaphoreType.DMA,
        ],
    )
    def k(x_hbm, src_hbm, dst_hbm, z_hbm, out_hbm,
          src_v, dst_v, srcx_v, dstx_v, rows_v, agg_s, sem):
        cid = lax.axis_index("c")
        sid = lax.axis_index("s")
        wid = sid * NC + cid
        r0 = sid * RPT
        pltpu.sync_copy(z_hbm.at[pl.ds(r0, RPT)], agg_s.at[pl.ds(r0, RPT)])
        pltpu.sync_copy(src_hbm.at[wid], src_v)
        pltpu.sync_copy(dst_hbm.at[wid], dst_v)

        @pl.when(cid == 0)
        def _():
            pltpu.sync_copy(src_hbm.at[wid + 1, pl.ds(CPT - STEAL, STEAL)],
                            srcx_v)
            pltpu.sync_copy(dst_hbm.at[wid + 1, pl.ds(CPT - STEAL, STEAL)],
                            dstx_v)

        plsc.subcore_barrier()

        @pl.loop(0, CPT - STEAL)
        def _(j):
            pltpu.async_copy(x_hbm.at[src_v.at[j]], rows_v, sem).wait()
            pltpu.sync_copy(rows_v, agg_s.at[dst_v.at[j]], add=True)

        @pl.when(cid == 0)
        def _():
            @pl.loop(CPT - STEAL, CPT)
            def _(j):
                pltpu.async_copy(x_hbm.at[src_v.at[j]], rows_v, sem).wait()
                pltpu.sync_copy(rows_v, agg_s.at[dst_v.at[j]], add=True)

            @pl.loop(0, STEAL)
            def _(j):
                pltpu.async_copy(x_hbm.at[srcx_v.at[j]], rows_v, sem).wait()
                pltpu.sync_copy(rows_v, agg_s.at[dstx_v.at[j]], add=True)

        plsc.subcore_barrier()
        pltpu.sync_copy(agg_s.at[pl.ds(r0, RPT)],
                        out_hbm.at[cid, pl.ds(r0, RPT)])

    return k(x, src_t, dst_t, zeros)


def _tc_layer(p0, p1, x, W_relT, b8, W_rootT, do_relu):
    BLK = 1000

    def body(p0_ref, p1_ref, x_ref, wr_ref, b_ref, wt_ref, o_ref):
        agg = p0_ref[...] + p1_ref[...]
        acc = jnp.dot(agg, wr_ref[...], preferred_element_type=jnp.float32)
        acc = acc + jnp.dot(x_ref[...], wt_ref[...],
                            preferred_element_type=jnp.float32)
        acc = acc + b_ref[0:1, :]
        if do_relu:
            acc = jnp.maximum(acc, 0.0)
        o_ref[...] = acc

    return pl.pallas_call(
        body,
        grid=(N // BLK,),
        in_specs=[
            pl.BlockSpec((BLK, D), lambda i: (i, 0)),
            pl.BlockSpec((BLK, D), lambda i: (i, 0)),
            pl.BlockSpec((BLK, D), lambda i: (i, 0)),
            pl.BlockSpec((D, D), lambda i: (0, 0)),
            pl.BlockSpec((8, D), lambda i: (0, 0)),
            pl.BlockSpec((D, D), lambda i: (0, 0)),
        ],
        out_specs=pl.BlockSpec((BLK, D), lambda i: (i, 0)),
        out_shape=jax.ShapeDtypeStruct((N, D), jnp.float32),
    )(p0, p1, x, W_relT, b8, W_rootT)


def kernel(h, edge_index, edge_attr, W_rel1, b_rel1, W_root1, W_rel2, b_rel2, W_root2):
    src = edge_index[0]
    dst = edge_index[1]
    pad = E_PAD - E
    src_t = jnp.concatenate(
        [src, jnp.zeros((pad,), jnp.int32)]).reshape(NW, CPT, CHUNK)
    dst_t = jnp.concatenate(
        [dst, jnp.full((pad,), N, jnp.int32)]).reshape(NW, CPT, CHUNK)
    zeros = jnp.zeros((N_PAD, D), jnp.float32)
    b1 = jnp.broadcast_to(b_rel1.reshape(1, D), (8, D))
    b2 = jnp.broadcast_to(b_rel2.reshape(1, D), (8, D))

    p = _sc_agg(h, src_t, dst_t, zeros)
    x1 = _tc_layer(p[0, :N], p[1, :N], h, W_rel1.T, b1, W_root1.T, True)
    p2 = _sc_agg(x1, src_t, dst_t, zeros)
    out = _tc_layer(p2[0, :N], p2[1, :N], x1, W_rel2.T, b2, W_root2.T, False)
    return out


# speedup vs baseline: 1.6080x; 1.2813x over previous
import functools

import jax
import jax.numpy as jnp
from jax import lax
from jax.experimental import pallas as pl
from jax.experimental.pallas import tpu as pltpu
from jax.experimental.pallas import tpu_sc as plsc

N = 10000
E = 320000
D = 128

NC = 2
NS = 16
NW = NC * NS
CHUNK = 128
CPT = 79
EPT = CHUNK * CPT
E_PAD = EPT * NW
N_PAD = 10112
RPT = N_PAD // NS


def _sc_agg(x, src_t, dst_t, zeros):
    mesh = plsc.VectorSubcoreMesh(core_axis_name="c", subcore_axis_name="s")

    @functools.partial(
        pl.kernel,
        out_type=jax.ShapeDtypeStruct((NC, N_PAD, D), jnp.float32),
        mesh=mesh,
        scratch_types=[
            pltpu.VMEM((CPT, CHUNK), jnp.int32),
            pltpu.VMEM((CPT, CHUNK), jnp.int32),
            pltpu.VMEM((CHUNK, D), jnp.float32),
            pltpu.VMEM_SHARED((N_PAD, D), jnp.float32),
            pltpu.SemaphoreType.DMA,
        ],
    )
    def k(x_hbm, src_hbm, dst_hbm, z_hbm, out_hbm, src_v, dst_v, rows_v, agg_s, sem):
        cid = lax.axis_index("c")
        sid = lax.axis_index("s")
        wid = sid * NC + cid
        r0 = sid * RPT
        pltpu.sync_copy(z_hbm.at[pl.ds(r0, RPT)], agg_s.at[pl.ds(r0, RPT)])
        pltpu.sync_copy(src_hbm.at[wid], src_v)
        pltpu.sync_copy(dst_hbm.at[wid], dst_v)
        plsc.subcore_barrier()

        @pl.loop(0, CPT)
        def _(j):
            pltpu.async_copy(x_hbm.at[src_v.at[j]], rows_v, sem).wait()
            pltpu.sync_copy(rows_v, agg_s.at[dst_v.at[j]], add=True)

        plsc.subcore_barrier()
        pltpu.sync_copy(agg_s.at[pl.ds(r0, RPT)],
                        out_hbm.at[cid, pl.ds(r0, RPT)])

    return k(x, src_t, dst_t, zeros)


def _tc_layer(p0, p1, x, W_relT, b8, W_rootT, do_relu):
    BLK = 1000

    def body(p0_ref, p1_ref, x_ref, wr_ref, b_ref, wt_ref, o_ref):
        agg = p0_ref[...] + p1_ref[...]
        acc = jnp.dot(agg, wr_ref[...], preferred_element_type=jnp.float32)
        acc = acc + jnp.dot(x_ref[...], wt_ref[...],
                            preferred_element_type=jnp.float32)
        acc = acc + b_ref[0:1, :]
        if do_relu:
            acc = jnp.maximum(acc, 0.0)
        o_ref[...] = acc

    return pl.pallas_call(
        body,
        grid=(N // BLK,),
        in_specs=[
            pl.BlockSpec((BLK, D), lambda i: (i, 0)),
            pl.BlockSpec((BLK, D), lambda i: (i, 0)),
            pl.BlockSpec((BLK, D), lambda i: (i, 0)),
            pl.BlockSpec((D, D), lambda i: (0, 0)),
            pl.BlockSpec((8, D), lambda i: (0, 0)),
            pl.BlockSpec((D, D), lambda i: (0, 0)),
        ],
        out_specs=pl.BlockSpec((BLK, D), lambda i: (i, 0)),
        out_shape=jax.ShapeDtypeStruct((N, D), jnp.float32),
    )(p0, p1, x, W_relT, b8, W_rootT)


def kernel(h, edge_index, edge_attr, W_rel1, b_rel1, W_root1, W_rel2, b_rel2, W_root2):
    src = edge_index[0]
    dst = edge_index[1]
    pad = E_PAD - E
    src_t = jnp.concatenate(
        [src, jnp.zeros((pad,), jnp.int32)]).reshape(NW, CPT, CHUNK)
    dst_t = jnp.concatenate(
        [dst, jnp.full((pad,), N, jnp.int32)]).reshape(NW, CPT, CHUNK)
    zeros = jnp.zeros((N_PAD, D), jnp.float32)
    b1 = jnp.broadcast_to(b_rel1.reshape(1, D), (8, D))
    b2 = jnp.broadcast_to(b_rel2.reshape(1, D), (8, D))

    p = _sc_agg(h, src_t, dst_t, zeros)
    x1 = _tc_layer(p[0, :N], p[1, :N], h, W_rel1.T, b1, W_root1.T, True)
    p2 = _sc_agg(x1, src_t, dst_t, zeros)
    out = _tc_layer(p2[0, :N], p2[1, :N], x1, W_rel2.T, b2, W_root2.T, False)
    return out


# speedup vs baseline: 1.6887x; 1.0502x over previous
import functools

import jax
import jax.numpy as jnp
from jax import lax
from jax.experimental import pallas as pl
from jax.experimental.pallas import tpu as pltpu
from jax.experimental.pallas import tpu_sc as plsc

N = 10000
E = 320000
D = 128

NC = 2
NS = 16
NW = NC * NS
CHUNK = 128
CPT = 79
EPT = CHUNK * CPT
E_PAD = EPT * NW
N_PAD = 10112
RPT = N_PAD // NS


def _sc_agg(x, src_t, dst_t, zeros):
    mesh = plsc.VectorSubcoreMesh(core_axis_name="c", subcore_axis_name="s")

    @functools.partial(
        pl.kernel,
        out_type=jax.ShapeDtypeStruct((NC, N_PAD, D), jnp.float32),
        mesh=mesh,
        scratch_types=[
            pltpu.VMEM((CPT, CHUNK), jnp.int32),
            pltpu.VMEM((CPT, CHUNK), jnp.int32),
            pltpu.VMEM((CHUNK, D), jnp.float32),
            pltpu.VMEM_SHARED((N_PAD, D), jnp.float32),
            pltpu.SemaphoreType.DMA,
        ],
    )
    def k(x_hbm, src_hbm, dst_hbm, z_hbm, out_hbm, src_v, dst_v, rows_v, agg_s, sem):
        cid = lax.axis_index("c")
        sid = lax.axis_index("s")
        wid = sid * NC + cid
        r0 = sid * RPT
        pltpu.sync_copy(z_hbm.at[pl.ds(r0, RPT)], agg_s.at[pl.ds(r0, RPT)])
        pltpu.sync_copy(src_hbm.at[wid], src_v)
        pltpu.sync_copy(dst_hbm.at[wid], dst_v)
        plsc.subcore_barrier()

        @pl.loop(0, CPT)
        def _(j):
            pltpu.async_copy(x_hbm.at[src_v.at[j]], rows_v, sem).wait()
            pltpu.sync_copy(rows_v, agg_s.at[dst_v.at[j]], add=True)

        plsc.subcore_barrier()
        pltpu.sync_copy(agg_s.at[pl.ds(r0, RPT)],
                        out_hbm.at[cid, pl.ds(r0, RPT)])

    return k(x, src_t, dst_t, zeros)


def _tc_root(x, W_rootT, b8):
    BLK = 1000

    def body(x_ref, wt_ref, b_ref, o_ref):
        acc = jnp.dot(x_ref[...], wt_ref[...],
                      preferred_element_type=jnp.float32)
        o_ref[...] = acc + b_ref[0:1, :]

    return pl.pallas_call(
        body,
        grid=(N // BLK,),
        in_specs=[
            pl.BlockSpec((BLK, D), lambda i: (i, 0)),
            pl.BlockSpec((D, D), lambda i: (0, 0)),
            pl.BlockSpec((8, D), lambda i: (0, 0)),
        ],
        out_specs=pl.BlockSpec((BLK, D), lambda i: (i, 0)),
        out_shape=jax.ShapeDtypeStruct((N, D), jnp.float32),
    )(x, W_rootT, b8)


def _tc_rel(p, r, W_relT, do_relu):
    BLK = 1000

    def body(p0_ref, p1_ref, r_ref, wr_ref, o_ref):
        agg = p0_ref[0] + p1_ref[0]
        acc = jnp.dot(agg, wr_ref[...], preferred_element_type=jnp.float32)
        acc = acc + r_ref[...]
        if do_relu:
            acc = jnp.maximum(acc, 0.0)
        o_ref[...] = acc

    return pl.pallas_call(
        body,
        grid=(N // BLK,),
        in_specs=[
            pl.BlockSpec((1, BLK, D), lambda i: (0, i, 0)),
            pl.BlockSpec((1, BLK, D), lambda i: (1, i, 0)),
            pl.BlockSpec((BLK, D), lambda i: (i, 0)),
            pl.BlockSpec((D, D), lambda i: (0, 0)),
        ],
        out_specs=pl.BlockSpec((BLK, D), lambda i: (i, 0)),
        out_shape=jax.ShapeDtypeStruct((N, D), jnp.float32),
    )(p, p, r, W_relT)


def kernel(h, edge_index, edge_attr, W_rel1, b_rel1, W_root1, W_rel2, b_rel2, W_root2):
    src = edge_index[0]
    dst = edge_index[1]
    pad = E_PAD - E
    src_t = jnp.concatenate(
        [src, jnp.zeros((pad,), jnp.int32)]).reshape(NW, CPT, CHUNK)
    dst_t = jnp.concatenate(
        [dst, jnp.full((pad,), N, jnp.int32)]).reshape(NW, CPT, CHUNK)
    zeros = jnp.zeros((N_PAD, D), jnp.float32)
    b1 = jnp.broadcast_to(b_rel1.reshape(1, D), (8, D))
    b2 = jnp.broadcast_to(b_rel2.reshape(1, D), (8, D))

    p = _sc_agg(h, src_t, dst_t, zeros)
    r1 = _tc_root(h, W_root1.T, b1)
    x1 = _tc_rel(p, r1, W_rel1.T, True)
    p2 = _sc_agg(x1, src_t, dst_t, zeros)
    r2 = _tc_root(x1, W_root2.T, b2)
    out = _tc_rel(p2, r2, W_rel2.T, False)
    return out
